# baseline (speedup 1.0000x reference)
import functools

import jax
import jax.numpy as jnp
from jax import lax
from jax.experimental import pallas as pl
from jax.experimental.pallas import tpu as pltpu
from jax.experimental.pallas import tpu_sc as plsc

BATCH = 16384
EMBED_DIM = 32
NUM_ROWS = 1000000
CHUNK = 128


def _make_kernel(num_cores, num_workers, b_per_w):
    mesh = plsc.VectorSubcoreMesh(core_axis_name="c", subcore_axis_name="s")
    n_md_chunks = b_per_w // CHUNK
    n_el = b_per_w * EMBED_DIM
    n_el_chunks = n_el // CHUNK

    @functools.partial(
        pl.kernel,
        mesh=mesh,
        compiler_params=pltpu.CompilerParams(needs_layout_passes=False),
        out_type=jax.ShapeDtypeStruct((BATCH,), jnp.float32),
        scratch_types=[
            pltpu.VMEM((b_per_w,), jnp.int32),
            pltpu.VMEM((b_per_w,), jnp.int32),
            pltpu.VMEM((n_el,), jnp.int32),
            pltpu.VMEM((n_el,), jnp.int32),
            pltpu.VMEM((n_el,), jnp.float32),
            pltpu.VMEM((n_el,), jnp.float32),
            pltpu.VMEM((b_per_w,), jnp.float32),
            pltpu.VMEM((b_per_w,), jnp.float32),
            pltpu.VMEM((b_per_w,), jnp.float32),
            pltpu.SemaphoreType.DMA,
            pltpu.SemaphoreType.DMA,
        ],
    )
    def sc_kernel(i_hbm, j_hbm, m_hbm, d_hbm, ut_hbm, vt_hbm, out_hbm,
                  idx_i, idx_j, idx_u, idx_v, u_vals, v_vals, m_v, d_v,
                  out_v, sem, sem_md):
        wid = lax.axis_index("s") * num_cores + lax.axis_index("c")
        base = wid * b_per_w

        pltpu.sync_copy(i_hbm.at[pl.ds(base, b_per_w)], idx_i)
        pltpu.sync_copy(j_hbm.at[pl.ds(base, b_per_w)], idx_j)

        md_copies = []
        for c in range(n_md_chunks):
            s = pl.ds(c * CHUNK, CHUNK)
            md_copies.append(
                pltpu.async_copy(m_hbm.at[idx_i.at[s]], m_v.at[s], sem_md))
            md_copies.append(
                pltpu.async_copy(d_hbm.at[idx_j.at[s]], d_v.at[s], sem_md))

        def build_body(g, carry):
            gb = g * 16
            iv = idx_i[pl.ds(gb, 16)]
            jv = idx_j[pl.ds(gb, 16)]
            for k in range(EMBED_DIM):
                off = k * NUM_ROWS
                idx_u[pl.ds(k * b_per_w + gb, 16)] = iv + off
                idx_v[pl.ds(k * b_per_w + gb, 16)] = jv + off
            return carry

        lax.fori_loop(0, b_per_w // 16, build_body, 0)

        for c in range(n_el_chunks):
            s = pl.ds(c * CHUNK, CHUNK)
            pltpu.async_copy(ut_hbm.at[idx_u.at[s]], u_vals.at[s], sem)
            pltpu.async_copy(vt_hbm.at[idx_v.at[s]], v_vals.at[s], sem)

        for cp in md_copies:
            cp.wait()
        pltpu.make_async_copy(
            ut_hbm.at[pl.ds(0, n_el)], u_vals, sem).wait()
        pltpu.make_async_copy(
            vt_hbm.at[pl.ds(0, n_el)], v_vals, sem).wait()

        def group_body(g, carry):
            gb = g * 16
            acc = m_v[pl.ds(gb, 16)] + d_v[pl.ds(gb, 16)]
            for k in range(EMBED_DIM):
                acc = acc + (u_vals[pl.ds(k * b_per_w + gb, 16)]
                             * v_vals[pl.ds(k * b_per_w + gb, 16)])
            out_v[pl.ds(gb, 16)] = acc
            return carry

        lax.fori_loop(0, b_per_w // 16, group_body, 0)

        pltpu.sync_copy(out_v, out_hbm.at[pl.ds(base, b_per_w)])

    return sc_kernel


def kernel(ij, m_bar, d_bar, U, V):
    i = jnp.asarray(ij[:, 0], dtype=jnp.int32)
    j = jnp.asarray(ij[:, 1], dtype=jnp.int32)
    info = plsc.get_sparse_core_info()
    num_workers = info.num_cores * info.num_subcores
    b_per_w = BATCH // num_workers
    ut = U.T.reshape(-1)
    vt = V.T.reshape(-1)
    return _make_kernel(info.num_cores, num_workers, b_per_w)(
        i, j, m_bar, d_bar, ut, vt)
